# Initial kernel scaffold; baseline (speedup 1.0000x reference)
import jax
import jax.numpy as jnp
from jax.experimental import pallas as pl


def kernel(x, edge_index, edge_attr, batch, r_target, Wq1, bq1, Wk1, bk1, Wv1, bv1, We1, Ws1, bs1, Wq2, bq2, Wk2, bk2, Wv2, bv2, We2, Ws2, bs2, Wq3, bq3, Wk3, bk3, Wv3, bv3, We3, Ws3, bs3, Wh, bh):
    raise NotImplementedError("write your pallas kernel here")


# speedup vs baseline: 3.8671x; 3.8671x over previous
import functools
import numpy as np
import jax
import jax.numpy as jnp
from jax import lax
from jax.experimental import pallas as pl
from jax.experimental.pallas import tpu as pltpu
from jax.experimental.pallas import tpu_sc as plsc

_N = 10000
_E = 320000
_D = 128
_H = 4
_C = 32
_ED = 4
_G = 128
_T = 4
_HC = _H * _C
_HE = _H * _ED
_ROW = 160
_NC = 2
_NS = 16
_NW = _NC * _NS
_CH = 32
_NCHUNK = _E // _CH
_TMAX = (_NCHUNK + _NW - 1) // _NW
_NP = 10240
_RPT = _NP // _NS
_INV_SQRT_C = float(1.0 / np.sqrt(_C))
_RB = 1000
_NBLK = _N // _RB

_WP_MASK = np.zeros((_HC, _HE), np.float32)
for _h in range(_H):
    _WP_MASK[_h * _C:(_h + 1) * _C, _h * _ED:(_h + 1) * _ED] = 1.0
_WP_MASK_J = _WP_MASK

_DEN_ONES = np.zeros((_H, _HC), np.float32)
for _h in range(_H):
    _DEN_ONES[_h, _h * _C:(_h + 1) * _C] = 1.0
_DEN_ONES_J = _DEN_ONES


def _proj_body(x_ref, wall_ref, ball_ref, wp_ref, qp_ref, kv_ref, s_ref):
    x = x_ref[...]
    qkvs = jnp.dot(x, wall_ref[...], preferred_element_type=jnp.float32, precision=lax.Precision.HIGHEST)
    qkvs = qkvs + ball_ref[0:1, :]
    q = qkvs[:, 0:_HC]
    p = jnp.dot(q, wp_ref[...], preferred_element_type=jnp.float32, precision=lax.Precision.HIGHEST)
    qp_ref[:, 0:_HC] = q
    qp_ref[:, _HC:_HC + _HE] = p
    kv_ref[...] = qkvs[:, _HC:3 * _HC]
    s_ref[...] = qkvs[:, 3 * _HC:4 * _HC]


def _proj(x, wall, ball8, wp):
    return pl.pallas_call(
        _proj_body,
        grid=(_NBLK,),
        in_specs=[
            pl.BlockSpec((_RB, _D), lambda i: (i, 0)),
            pl.BlockSpec((_D, 4 * _HC), lambda i: (0, 0)),
            pl.BlockSpec((8, 4 * _HC), lambda i: (0, 0)),
            pl.BlockSpec((_HC, _HE), lambda i: (0, 0)),
        ],
        out_specs=[
            pl.BlockSpec((_RB, _HC + _HE), lambda i: (i, 0)),
            pl.BlockSpec((_RB, 2 * _HC), lambda i: (i, 0)),
            pl.BlockSpec((_RB, _HC), lambda i: (i, 0)),
        ],
        out_shape=[
            jax.ShapeDtypeStruct((_N, _HC + _HE), jnp.float32),
            jax.ShapeDtypeStruct((_N, 2 * _HC), jnp.float32),
            jax.ShapeDtypeStruct((_N, _HC), jnp.float32),
        ],
    )(x, wall, ball8, wp)


def _combine_body(acc_ref, s_ref, wse_ref, dones_ref, out_ref, *, relu):
    a = acc_ref[0] + acc_ref[1]
    num = a[:, 0:_HC]
    den4 = a[:, _HC:_HC + _H]
    s4 = a[:, _HC + 16:_ROW]
    eterm = jnp.dot(s4, wse_ref[...], preferred_element_type=jnp.float32, precision=lax.Precision.HIGHEST)
    den_rep = jnp.dot(den4, dones_ref[...], preferred_element_type=jnp.float32, precision=lax.Precision.HIGHEST)
    h = (num + eterm) / (den_rep + 1e-16) + s_ref[...]
    if relu:
        h = jnp.maximum(h, 0.0)
    out_ref[...] = h


def _combine(acc, s, wse, relu):
    return pl.pallas_call(
        functools.partial(_combine_body, relu=relu),
        grid=(_NBLK,),
        in_specs=[
            pl.BlockSpec((_NC, _RB, _ROW), lambda i: (0, i, 0)),
            pl.BlockSpec((_RB, _HC), lambda i: (i, 0)),
            pl.BlockSpec((_HE, _HC), lambda i: (0, 0)),
            pl.BlockSpec((_H, _HC), lambda i: (0, 0)),
        ],
        out_specs=pl.BlockSpec((_RB, _HC), lambda i: (i, 0)),
        out_shape=jax.ShapeDtypeStruct((_N, _HC), jnp.float32),
    )(acc, s, wse, _DEN_ONES_J)


def _pool_body(h_ref, b_ref, wh_ref, bh_ref, out_ref, acc_ref, cnt_ref):
    i = pl.program_id(0)

    @pl.when(i == 0)
    def _init():
        acc_ref[...] = jnp.zeros_like(acc_ref)
        cnt_ref[...] = jnp.zeros_like(cnt_ref)

    b = b_ref[...]
    gid = lax.broadcasted_iota(jnp.int32, (_RB, _G), 1).astype(jnp.float32)
    onehot = (b == gid).astype(jnp.float32)
    acc_ref[...] += lax.dot_general(onehot, h_ref[...],
                                    (((0,), (0,)), ((), ())),
                                    preferred_element_type=jnp.float32,
                                    precision=lax.Precision.HIGHEST)
    cnt_ref[...] += jnp.sum(onehot, axis=0, keepdims=True)

    @pl.when(i == _NBLK - 1)
    def _final():
        cnt = jnp.maximum(cnt_ref[...], 1.0)
        pooled = acc_ref[...] / cnt.reshape(_G, 1)
        out_ref[...] = jnp.dot(pooled, wh_ref[...],
                               preferred_element_type=jnp.float32, precision=lax.Precision.HIGHEST) + bh_ref[0:1, :]


def _pool(h, b2d, wh, bh8):
    return pl.pallas_call(
        _pool_body,
        grid=(_NBLK,),
        in_specs=[
            pl.BlockSpec((_RB, _HC), lambda i: (i, 0)),
            pl.BlockSpec((_RB, 1), lambda i: (i, 0)),
            pl.BlockSpec((_HC, _T), lambda i: (0, 0)),
            pl.BlockSpec((8, _T), lambda i: (0, 0)),
        ],
        out_specs=pl.BlockSpec((_G, _T), lambda i: (0, 0)),
        out_shape=jax.ShapeDtypeStruct((_G, _T), jnp.float32),
        scratch_shapes=[
            pltpu.VMEM((_G, _HC), jnp.float32),
            pltpu.VMEM((1, _G), jnp.float32),
        ],
    )(h, b2d, wh, bh8)


def _edge_sc_body(qp_hbm, kv_hbm, di_hbm, si_hbm, ea_hbm, z_hbm, acc_out,
                  di_v, si_v, ea_v, qp_v, kv_v, msg_v, acc_sh, sem1, sem2):
    cid = lax.axis_index("c")
    sid = lax.axis_index("s")
    wid = sid * _NC + cid
    row0 = sid * _RPT

    pltpu.sync_copy(z_hbm.at[pl.ds(row0, _RPT)], acc_sh.at[pl.ds(row0, _RPT)])
    plsc.subcore_barrier()

    li = lax.iota(jnp.int32, 16)

    def chunk_body(t, carry):
        cix = wid + t * _NW

        @pl.when(cix < _NCHUNK)
        def _():
            base = cix * _CH
            pltpu.sync_copy(di_hbm.at[pl.ds(base, _CH)], di_v)
            pltpu.sync_copy(si_hbm.at[pl.ds(base, _CH)], si_v)
            pltpu.sync_copy(ea_hbm.at[pl.ds(base, _CH)], ea_v)
            cp1 = pltpu.async_copy(qp_hbm.at[di_v], qp_v, sem1)
            cp2 = pltpu.async_copy(kv_hbm.at[si_v], kv_v, sem2)
            cp1.wait()
            cp2.wait()

            def edge(i, c2):
                ea_row = ea_v[i, :]
                p_row = qp_v[i, pl.ds(_HC, 16)]
                pterm = p_row * ea_row
                exs = []
                for h in range(_H):
                    sh = (qp_v[i, pl.ds(2 * h * 16, 16)] * kv_v[i, pl.ds(2 * h * 16, 16)]
                          + qp_v[i, pl.ds((2 * h + 1) * 16, 16)] * kv_v[i, pl.ds((2 * h + 1) * 16, 16)])
                    qk_s = jnp.sum(sh)
                    in_h = (li >= 4 * h) & (li < 4 * h + 4)
                    pe_s = jnp.sum(jnp.where(in_h, pterm, 0.0))
                    alpha_s = (qk_s + pe_s) * _INV_SQRT_C
                    exs.append(jnp.exp(jnp.full((16,), alpha_s, jnp.float32)))
                for j in range(8):
                    msg_v[i, pl.ds(j * 16, 16)] = (
                        kv_v[i, pl.ds(_HC + j * 16, 16)] * exs[j // 2])
                dv = jnp.where(
                    li == 0, exs[0],
                    jnp.where(li == 1, exs[1],
                              jnp.where(li == 2, exs[2],
                                        jnp.where(li == 3, exs[3], 0.0))))
                exrep = jnp.where(
                    li < 4, exs[0],
                    jnp.where(li < 8, exs[1],
                              jnp.where(li < 12, exs[2], exs[3])))
                msg_v[i, pl.ds(_HC, 16)] = dv
                msg_v[i, pl.ds(_HC + 16, 16)] = exrep * ea_row
                return c2

            lax.fori_loop(0, _CH, edge, 0)
            pltpu.sync_copy(msg_v, acc_sh.at[di_v], add=True)

        return carry

    lax.fori_loop(0, _TMAX, chunk_body, 0)
    plsc.subcore_barrier()
    pltpu.sync_copy(acc_sh.at[pl.ds(row0, _RPT)],
                    acc_out.at[cid, pl.ds(row0, _RPT)])


@functools.cache
def _edge_kernel_built():
    mesh = plsc.VectorSubcoreMesh(core_axis_name="c", subcore_axis_name="s",
                                  num_cores=_NC, num_subcores=_NS)
    return pl.kernel(
        _edge_sc_body,
        out_type=jax.ShapeDtypeStruct((_NC, _NP, _ROW), jnp.float32),
        mesh=mesh,
        compiler_params=pltpu.CompilerParams(needs_layout_passes=False,
                                             use_tc_tiling_on_sc=False),
        scratch_types=[
            pltpu.VMEM((_CH,), jnp.int32),
            pltpu.VMEM((_CH,), jnp.int32),
            pltpu.VMEM((_CH, _HE), jnp.float32),
            pltpu.VMEM((_CH, _HC + _HE), jnp.float32),
            pltpu.VMEM((_CH, 2 * _HC), jnp.float32),
            pltpu.VMEM((_CH, _ROW), jnp.float32),
            pltpu.VMEM_SHARED((_NP, _ROW), jnp.float32),
            pltpu.SemaphoreType.DMA,
            pltpu.SemaphoreType.DMA,
        ],
    )


def _edge_kernel(*args):
    return _edge_kernel_built()(*args)


def _layer(h, di, si, ea2, z, wq, bq, wk, bk, wv, bv, we, ws, bs, relu):
    wall = jnp.concatenate([wq, wk, wv, ws], axis=1)
    ball8 = jnp.broadcast_to(
        jnp.concatenate([bq, bk, bv, bs])[None, :], (8, 4 * _HC))
    wp = _WP_MASK_J * jnp.tile(we.T, (1, _H))
    qp, kv, s = _proj(h, wall, ball8, wp)
    acc = _edge_kernel(qp, kv, di, si, ea2, z)
    return _combine(acc, s, wp.T, relu)


def kernel(x, edge_index, edge_attr, batch, r_target,
           Wq1, bq1, Wk1, bk1, Wv1, bv1, We1, Ws1, bs1,
           Wq2, bq2, Wk2, bk2, Wv2, bv2, We2, Ws2, bs2,
           Wq3, bq3, Wk3, bk3, Wv3, bv3, We3, Ws3, bs3, Wh, bh):
    si = edge_index[0]
    di = edge_index[1]
    ea2 = jnp.tile(edge_attr, (1, _H))
    z = jnp.zeros((_NP, _ROW), jnp.float32)
    h = _layer(x, di, si, ea2, z, Wq1, bq1, Wk1, bk1, Wv1, bv1, We1, Ws1, bs1, True)
    h = _layer(h, di, si, ea2, z, Wq2, bq2, Wk2, bk2, Wv2, bv2, We2, Ws2, bs2, True)
    h = _layer(h, di, si, ea2, z, Wq3, bq3, Wk3, bk3, Wv3, bv3, We3, Ws3, bs3, False)
    b2d = batch.astype(jnp.float32).reshape(_N, 1)
    bh8 = jnp.broadcast_to(bh[None, :], (8, _T))
    return _pool(h, b2d, Wh, bh8)


# speedup vs baseline: 7.1343x; 1.8449x over previous
import functools
import numpy as np
import jax
import jax.numpy as jnp
from jax import lax
from jax.experimental import pallas as pl
from jax.experimental.pallas import tpu as pltpu
from jax.experimental.pallas import tpu_sc as plsc

_N = 10000
_E = 320000
_D = 128
_H = 4
_C = 32
_ED = 4
_G = 128
_T = 4
_HC = _H * _C
_HE = _H * _ED
_ROW = 160
_NC = 2
_NS = 16
_NW = _NC * _NS
_NP = 10112
_RPT = _NP // _NS
_CH = 24
_EP = 322560
_NCHUNK = _EP // _CH
_CPW = _NCHUNK // _NW
_SB = 6
_OUTER = _CPW // (2 * _SB)
_INV_SQRT_C = float(1.0 / np.sqrt(_C))
_RB = 1000
_PRB = 632
_NBLK = _N // _RB

_WP_MASK = np.zeros((_HC, _HE), np.float32)
for _h in range(_H):
    _WP_MASK[_h * _C:(_h + 1) * _C, _h * _ED:(_h + 1) * _ED] = 1.0

_DEN_ONES2 = np.zeros((_HE, _HC), np.float32)
for _h in range(_H):
    _DEN_ONES2[4 * _h, _h * _C:(_h + 1) * _C] = 1.0


def _proj_body(x_ref, wall_ref, ball_ref, wp_ref, qp_ref, kv_ref, s_ref):
    x = x_ref[...]
    qkvs = jnp.dot(x, wall_ref[...], preferred_element_type=jnp.float32,
                   precision=lax.Precision.HIGHEST)
    qkvs = qkvs + ball_ref[0:1, :]
    q = qkvs[:, 0:_HC]
    p = jnp.dot(q, wp_ref[...], preferred_element_type=jnp.float32,
                precision=lax.Precision.HIGHEST)
    qp_ref[:, 0:_HC] = q
    qp_ref[:, _HC:_HC + _HE] = p
    kv_ref[...] = qkvs[:, _HC:3 * _HC]
    s_ref[...] = qkvs[:, 3 * _HC:4 * _HC]


def _proj(x, wall, ball8, wp):
    return pl.pallas_call(
        _proj_body,
        grid=(_NS,),
        in_specs=[
            pl.BlockSpec((_PRB, _D), lambda i: (i, 0)),
            pl.BlockSpec((_D, 4 * _HC), lambda i: (0, 0)),
            pl.BlockSpec((8, 4 * _HC), lambda i: (0, 0)),
            pl.BlockSpec((_HC, _HE), lambda i: (0, 0)),
        ],
        out_specs=[
            pl.BlockSpec((_PRB, _HC + _HE), lambda i: (i, 0)),
            pl.BlockSpec((_PRB, 2 * _HC), lambda i: (i, 0)),
            pl.BlockSpec((_PRB, _HC), lambda i: (i, 0)),
        ],
        out_shape=[
            jax.ShapeDtypeStruct((_NP, _HC + _HE), jnp.float32),
            jax.ShapeDtypeStruct((_NP, 2 * _HC), jnp.float32),
            jax.ShapeDtypeStruct((_NP, _HC), jnp.float32),
        ],
    )(x, wall, ball8, wp)


def _combine_body(acc_ref, s_ref, wse_ref, dones_ref, out_ref, *, relu):
    a = acc_ref[0] + acc_ref[1]
    num = a[:, 0:_HC]
    exr = a[:, _HC:_HC + _HE]
    s4 = a[:, _HC + _HE:_ROW]
    eterm = jnp.dot(s4, wse_ref[...], preferred_element_type=jnp.float32,
                    precision=lax.Precision.HIGHEST)
    den_rep = jnp.dot(exr, dones_ref[...], preferred_element_type=jnp.float32,
                      precision=lax.Precision.HIGHEST)
    h = (num + eterm) / (den_rep + 1e-16) + s_ref[...]
    if relu:
        h = jnp.maximum(h, 0.0)
    out_ref[...] = h


def _combine(acc, s, wse, relu):
    return pl.pallas_call(
        functools.partial(_combine_body, relu=relu),
        grid=(_NS,),
        in_specs=[
            pl.BlockSpec((_NC, _PRB, _ROW), lambda i: (0, i, 0)),
            pl.BlockSpec((_PRB, _HC), lambda i: (i, 0)),
            pl.BlockSpec((_HE, _HC), lambda i: (0, 0)),
            pl.BlockSpec((_HE, _HC), lambda i: (0, 0)),
        ],
        out_specs=pl.BlockSpec((_PRB, _HC), lambda i: (i, 0)),
        out_shape=jax.ShapeDtypeStruct((_NP, _HC), jnp.float32),
    )(acc, s, wse, _DEN_ONES2)


def _pool_body(h_ref, b_ref, wh_ref, bh_ref, out_ref, acc_ref, cnt_ref):
    i = pl.program_id(0)

    @pl.when(i == 0)
    def _init():
        acc_ref[...] = jnp.zeros_like(acc_ref)
        cnt_ref[...] = jnp.zeros_like(cnt_ref)

    b = b_ref[...]
    gid = lax.broadcasted_iota(jnp.int32, (_RB, _G), 1).astype(jnp.float32)
    onehot = (b == gid).astype(jnp.float32)
    acc_ref[...] += lax.dot_general(onehot, h_ref[...],
                                    (((0,), (0,)), ((), ())),
                                    preferred_element_type=jnp.float32,
                                    precision=lax.Precision.HIGHEST)
    cnt_ref[...] += jnp.sum(onehot, axis=0, keepdims=True)

    @pl.when(i == _NBLK - 1)
    def _final():
        cnt = jnp.maximum(cnt_ref[...], 1.0)
        pooled = acc_ref[...] / cnt.reshape(_G, 1)
        out_ref[...] = jnp.dot(pooled, wh_ref[...],
                               preferred_element_type=jnp.float32,
                               precision=lax.Precision.HIGHEST) + bh_ref[0:1, :]


def _pool(h, b2d, wh, bh8):
    return pl.pallas_call(
        _pool_body,
        grid=(_NBLK,),
        in_specs=[
            pl.BlockSpec((_RB, _HC), lambda i: (i, 0)),
            pl.BlockSpec((_RB, 1), lambda i: (i, 0)),
            pl.BlockSpec((_HC, _T), lambda i: (0, 0)),
            pl.BlockSpec((8, _T), lambda i: (0, 0)),
        ],
        out_specs=pl.BlockSpec((_G, _T), lambda i: (0, 0)),
        out_shape=jax.ShapeDtypeStruct((_G, _T), jnp.float32),
        scratch_shapes=[
            pltpu.VMEM((_G, _HC), jnp.float32),
            pltpu.VMEM((1, _G), jnp.float32),
        ],
    )(h, b2d, wh, bh8)


def _edge_sc_body(qp_hbm, kv_hbm, di_hbm, si_hbm, ea_hbm, z_hbm, acc_out,
                  di_a, di_b, si_a, si_b, ea_a, ea_b,
                  qp0, qp1, kv0, kv1, msg_v, acc_sh,
                  sq0, sq1, sk0, sk1, sdi_a, sdi_b, ssi_a, ssi_b,
                  sea_a, sea_b):
    cid = lax.axis_index("c")
    sid = lax.axis_index("s")
    wid = sid * _NC + cid
    row0 = sid * _RPT
    c0 = wid * _CPW

    pltpu.sync_copy(z_hbm.at[pl.ds(row0, _RPT)], acc_sh.at[pl.ds(row0, _RPT)])
    plsc.subcore_barrier()

    li = lax.iota(jnp.int32, 16)
    in_h = [(li >= 4 * h) & (li < 4 * h + 4) for h in range(_H)]
    qps = (qp0, qp1)
    kvs = (kv0, kv1)
    sqs = (sq0, sq1)
    sks = (sk0, sk1)
    dis = (di_a, di_b)
    sis = (si_a, si_b)
    eas = (ea_a, ea_b)
    sbsems = ((sdi_a, ssi_a, sea_a), (sdi_b, ssi_b, sea_b))

    def load_sb_sync(sb_global, buf):
        g0 = sb_global * _SB
        pltpu.sync_copy(di_hbm.at[pl.ds(g0, _SB)], dis[buf])
        pltpu.sync_copy(si_hbm.at[pl.ds(g0, _SB)], sis[buf])
        pltpu.sync_copy(ea_hbm.at[pl.ds(g0 * _CH, _SB * _CH)], eas[buf])

    def load_sb_async(sb_global, buf):
        g0 = sb_global * _SB
        pltpu.async_copy(di_hbm.at[pl.ds(g0, _SB)], dis[buf], sbsems[buf][0])
        pltpu.async_copy(si_hbm.at[pl.ds(g0, _SB)], sis[buf], sbsems[buf][1])
        pltpu.async_copy(ea_hbm.at[pl.ds(g0 * _CH, _SB * _CH)], eas[buf], sbsems[buf][2])

    def wait_sb(sb_global, buf):
        g0 = sb_global * _SB
        pltpu.make_async_copy(di_hbm.at[pl.ds(g0, _SB)], dis[buf], sbsems[buf][0]).wait()
        pltpu.make_async_copy(si_hbm.at[pl.ds(g0, _SB)], sis[buf], sbsems[buf][1]).wait()
        pltpu.make_async_copy(ea_hbm.at[pl.ds(g0 * _CH, _SB * _CH)], eas[buf], sbsems[buf][2]).wait()

    def issue_gathers(sbbuf, j, gbuf):
        pltpu.async_copy(qp_hbm.at[dis[sbbuf].at[j]], qps[gbuf], sqs[gbuf])
        pltpu.async_copy(kv_hbm.at[sis[sbbuf].at[j]], kvs[gbuf], sks[gbuf])

    def wait_gathers(sbbuf, j, gbuf):
        pltpu.make_async_copy(qp_hbm.at[dis[sbbuf].at[j]], qps[gbuf], sqs[gbuf]).wait()
        pltpu.make_async_copy(kv_hbm.at[sis[sbbuf].at[j]], kvs[gbuf], sks[gbuf]).wait()

    def compute_chunk(sbbuf, j, gbuf):
        qp_v = qps[gbuf]
        kv_v = kvs[gbuf]
        ea_v = eas[sbbuf]

        def edge(i, c2):
            ea_row = ea_v[j * _CH + i, :]
            p_row = qp_v[i, pl.ds(_HC, 16)]
            pterm = p_row * ea_row
            exs = []
            for h in range(_H):
                sh = (qp_v[i, pl.ds(2 * h * 16, 16)] * kv_v[i, pl.ds(2 * h * 16, 16)]
                      + qp_v[i, pl.ds((2 * h + 1) * 16, 16)] * kv_v[i, pl.ds((2 * h + 1) * 16, 16)])
                sh = sh + jnp.where(in_h[h], pterm, 0.0)
                a_s = jnp.sum(sh)
                exs.append(jnp.exp(jnp.full((16,), a_s, jnp.float32)))
            for jj in range(8):
                msg_v[i, pl.ds(jj * 16, 16)] = (
                    kv_v[i, pl.ds(_HC + jj * 16, 16)] * exs[jj // 2])
            exrep = jnp.where(li < 4, exs[0],
                              jnp.where(li < 8, exs[1],
                                        jnp.where(li < 12, exs[2], exs[3])))
            msg_v[i, pl.ds(_HC, 16)] = exrep
            msg_v[i, pl.ds(_HC + _HE, 16)] = exrep * ea_row
            return c2

        lax.fori_loop(0, _CH, edge, 0)
        pltpu.sync_copy(msg_v, acc_sh.at[dis[sbbuf].at[j]], add=True)

    load_sb_sync(c0 // _SB, 0)
    issue_gathers(0, 0, 0)
    issue_gathers(0, 1, 1)

    def outer(s2, carry):
        sbA = c0 // _SB + 2 * s2
        sbB = sbA + 1
        u0 = s2 * 12
        for j in range(_SB):
            gbuf = j % 2
            if j == 0:
                load_sb_async(sbB, 1)
            if j == 4:
                wait_sb(sbB, 1)
            wait_gathers(0, j, gbuf)
            compute_chunk(0, j, gbuf)
            if j < _SB - 2:
                issue_gathers(0, j + 2, gbuf)
            else:
                issue_gathers(1, j + 2 - _SB, gbuf)
        for j in range(_SB):
            gbuf = j % 2
            if j == 0:
                @pl.when(s2 < _OUTER - 1)
                def _():
                    load_sb_async(sbA + 2, 0)
            if j == 4:
                @pl.when(s2 < _OUTER - 1)
                def _():
                    wait_sb(sbA + 2, 0)
            wait_gathers(1, j, gbuf)
            compute_chunk(1, j, gbuf)
            if j < _SB - 2:
                issue_gathers(1, j + 2, gbuf)
            else:
                @pl.when(s2 < _OUTER - 1)
                def _():
                    issue_gathers(0, j + 2 - _SB, gbuf)
        return carry

    lax.fori_loop(0, _OUTER, outer, 0)
    plsc.subcore_barrier()
    pltpu.sync_copy(acc_sh.at[pl.ds(row0, _RPT)],
                    acc_out.at[cid, pl.ds(row0, _RPT)])


@functools.cache
def _edge_kernel_built():
    mesh = plsc.VectorSubcoreMesh(core_axis_name="c", subcore_axis_name="s",
                                  num_cores=_NC, num_subcores=_NS)
    return pl.kernel(
        _edge_sc_body,
        out_type=jax.ShapeDtypeStruct((_NC, _NP, _ROW), jnp.float32),
        mesh=mesh,
        compiler_params=pltpu.CompilerParams(needs_layout_passes=False,
                                             use_tc_tiling_on_sc=False),
        scratch_types=[
            pltpu.VMEM((_SB, _CH), jnp.int32),
            pltpu.VMEM((_SB, _CH), jnp.int32),
            pltpu.VMEM((_SB, _CH), jnp.int32),
            pltpu.VMEM((_SB, _CH), jnp.int32),
            pltpu.VMEM((_SB * _CH, _HE), jnp.float32),
            pltpu.VMEM((_SB * _CH, _HE), jnp.float32),
            pltpu.VMEM((_CH, _HC + _HE), jnp.float32),
            pltpu.VMEM((_CH, _HC + _HE), jnp.float32),
            pltpu.VMEM((_CH, 2 * _HC), jnp.float32),
            pltpu.VMEM((_CH, 2 * _HC), jnp.float32),
            pltpu.VMEM((_CH, _ROW), jnp.float32),
            pltpu.VMEM_SHARED((_NP, _ROW), jnp.float32),
            pltpu.SemaphoreType.DMA,
            pltpu.SemaphoreType.DMA,
            pltpu.SemaphoreType.DMA,
            pltpu.SemaphoreType.DMA,
            pltpu.SemaphoreType.DMA,
            pltpu.SemaphoreType.DMA,
            pltpu.SemaphoreType.DMA,
            pltpu.SemaphoreType.DMA,
            pltpu.SemaphoreType.DMA,
            pltpu.SemaphoreType.DMA,
        ],
    )


def _edge_kernel(*args):
    return _edge_kernel_built()(*args)


def _layer(h, di2, si2, ea2, z, wq, bq, wk, bk, wv, bv, we, ws, bs, relu):
    wall = jnp.concatenate([wq * _INV_SQRT_C, wk, wv, ws], axis=1)
    ball8 = jnp.broadcast_to(
        jnp.concatenate([bq * _INV_SQRT_C, bk, bv, bs])[None, :], (8, 4 * _HC))
    wp = _WP_MASK * jnp.tile(we.T, (1, _H))
    qp, kv, s = _proj(h, wall, ball8, wp)
    acc = _edge_kernel(qp, kv, di2, si2, ea2, z)
    return _combine(acc, s, wp.T, relu)


def kernel(x, edge_index, edge_attr, batch, r_target,
           Wq1, bq1, Wk1, bk1, Wv1, bv1, We1, Ws1, bs1,
           Wq2, bq2, Wk2, bk2, Wv2, bv2, We2, Ws2, bs2,
           Wq3, bq3, Wk3, bk3, Wv3, bv3, We3, Ws3, bs3, Wh, bh):
    si = edge_index[0]
    di = edge_index[1]
    npad = _EP - _E
    di2 = jnp.concatenate([di, jnp.full((npad,), _NP - 1, jnp.int32)]).reshape(_NCHUNK, _CH)
    si2 = jnp.concatenate([si, jnp.zeros((npad,), jnp.int32)]).reshape(_NCHUNK, _CH)
    ea2 = jnp.concatenate([jnp.tile(edge_attr, (1, _H)),
                           jnp.zeros((npad, _HE), jnp.float32)])
    z = jnp.zeros((_NP, _ROW), jnp.float32)
    xp = jnp.concatenate([x, jnp.zeros((_NP - _N, _D), jnp.float32)])
    h = _layer(xp, di2, si2, ea2, z, Wq1, bq1, Wk1, bk1, Wv1, bv1, We1, Ws1, bs1, True)
    h = _layer(h, di2, si2, ea2, z, Wq2, bq2, Wk2, bk2, Wv2, bv2, We2, Ws2, bs2, True)
    h = _layer(h, di2, si2, ea2, z, Wq3, bq3, Wk3, bk3, Wv3, bv3, We3, Ws3, bs3, False)
    b2d = batch.astype(jnp.float32).reshape(_N, 1)
    bh8 = jnp.broadcast_to(bh[None, :], (8, _T))
    return _pool(h[:_N], b2d, Wh, bh8)


# speedup vs baseline: 9.2060x; 1.2904x over previous
import functools
import numpy as np
import jax
import jax.numpy as jnp
from jax import lax
from jax.experimental import pallas as pl
from jax.experimental.pallas import tpu as pltpu
from jax.experimental.pallas import tpu_sc as plsc

_N = 10000
_E = 320000
_D = 128
_H = 4
_C = 32
_ED = 4
_G = 128
_T = 4
_HC = _H * _C
_HE = _H * _ED
_ROW = 160
_NC = 2
_NS = 16
_NW = _NC * _NS
_NP = 10112
_RPT = _NP // _NS
_CH = 24
_EP = 322560
_NCHUNK = _EP // _CH
_CPW = _NCHUNK // _NW
_SB = 6
_OUTER = _CPW // (2 * _SB)
_INV_SQRT_C = float(1.0 / np.sqrt(_C))
_RB = 1000
_PRB = 632
_NBLK = _N // _RB

_WP_MASK = np.zeros((_HC, _HE), np.float32)
for _h in range(_H):
    _WP_MASK[_h * _C:(_h + 1) * _C, _h * _ED:(_h + 1) * _ED] = 1.0

_DEN_ONES2 = np.zeros((_HE, _HC), np.float32)
for _h in range(_H):
    _DEN_ONES2[4 * _h, _h * _C:(_h + 1) * _C] = 1.0


def _proj_body(x_ref, wall_ref, ball_ref, wp_ref, qp_ref, kv_ref, s_ref):
    x = x_ref[...]
    qkvs = jnp.dot(x, wall_ref[...], preferred_element_type=jnp.float32,
                   precision=lax.Precision.HIGHEST)
    qkvs = qkvs + ball_ref[0:1, :]
    q = qkvs[:, 0:_HC]
    p = jnp.dot(q, wp_ref[...], preferred_element_type=jnp.float32,
                precision=lax.Precision.HIGHEST)
    qp_ref[:, 0:_HC] = q
    qp_ref[:, _HC:_HC + _HE] = p
    kv_ref[...] = qkvs[:, _HC:3 * _HC]
    s_ref[...] = qkvs[:, 3 * _HC:4 * _HC]


def _proj(x, wall, ball8, wp):
    return pl.pallas_call(
        _proj_body,
        grid=(_NS,),
        in_specs=[
            pl.BlockSpec((_PRB, _D), lambda i: (i, 0)),
            pl.BlockSpec((_D, 4 * _HC), lambda i: (0, 0)),
            pl.BlockSpec((8, 4 * _HC), lambda i: (0, 0)),
            pl.BlockSpec((_HC, _HE), lambda i: (0, 0)),
        ],
        out_specs=[
            pl.BlockSpec((_PRB, _HC + _HE), lambda i: (i, 0)),
            pl.BlockSpec((_PRB, 2 * _HC), lambda i: (i, 0)),
            pl.BlockSpec((_PRB, _HC), lambda i: (i, 0)),
        ],
        out_shape=[
            jax.ShapeDtypeStruct((_NP, _HC + _HE), jnp.float32),
            jax.ShapeDtypeStruct((_NP, 2 * _HC), jnp.float32),
            jax.ShapeDtypeStruct((_NP, _HC), jnp.float32),
        ],
    )(x, wall, ball8, wp)


def _combine_body(acc_ref, s_ref, wse_ref, dones_ref, out_ref, *, relu):
    a = acc_ref[0] + acc_ref[1]
    num = a[:, 0:_HC]
    exr = a[:, _HC:_HC + _HE]
    s4 = a[:, _HC + _HE:_ROW]
    eterm = jnp.dot(s4, wse_ref[...], preferred_element_type=jnp.float32,
                    precision=lax.Precision.HIGHEST)
    den_rep = jnp.dot(exr, dones_ref[...], preferred_element_type=jnp.float32,
                      precision=lax.Precision.HIGHEST)
    h = (num + eterm) / (den_rep + 1e-16) + s_ref[...]
    if relu:
        h = jnp.maximum(h, 0.0)
    out_ref[...] = h


def _combine(acc, s, wse, relu):
    return pl.pallas_call(
        functools.partial(_combine_body, relu=relu),
        grid=(_NS,),
        in_specs=[
            pl.BlockSpec((_NC, _PRB, _ROW), lambda i: (0, i, 0)),
            pl.BlockSpec((_PRB, _HC), lambda i: (i, 0)),
            pl.BlockSpec((_HE, _HC), lambda i: (0, 0)),
            pl.BlockSpec((_HE, _HC), lambda i: (0, 0)),
        ],
        out_specs=pl.BlockSpec((_PRB, _HC), lambda i: (i, 0)),
        out_shape=jax.ShapeDtypeStruct((_NP, _HC), jnp.float32),
    )(acc, s, wse, _DEN_ONES2)


def _pool_body(h_ref, b_ref, wh_ref, bh_ref, out_ref, acc_ref, cnt_ref):
    i = pl.program_id(0)

    @pl.when(i == 0)
    def _init():
        acc_ref[...] = jnp.zeros_like(acc_ref)
        cnt_ref[...] = jnp.zeros_like(cnt_ref)

    b = b_ref[...]
    gid = lax.broadcasted_iota(jnp.int32, (_RB, _G), 1).astype(jnp.float32)
    onehot = (b == gid).astype(jnp.float32)
    acc_ref[...] += lax.dot_general(onehot, h_ref[...],
                                    (((0,), (0,)), ((), ())),
                                    preferred_element_type=jnp.float32,
                                    precision=lax.Precision.HIGHEST)
    cnt_ref[...] += jnp.sum(onehot, axis=0, keepdims=True)

    @pl.when(i == _NBLK - 1)
    def _final():
        cnt = jnp.maximum(cnt_ref[...], 1.0)
        pooled = acc_ref[...] / cnt.reshape(_G, 1)
        out_ref[...] = jnp.dot(pooled, wh_ref[...],
                               preferred_element_type=jnp.float32,
                               precision=lax.Precision.HIGHEST) + bh_ref[0:1, :]


def _pool(h, b2d, wh, bh8):
    return pl.pallas_call(
        _pool_body,
        grid=(_NBLK,),
        in_specs=[
            pl.BlockSpec((_RB, _HC), lambda i: (i, 0)),
            pl.BlockSpec((_RB, 1), lambda i: (i, 0)),
            pl.BlockSpec((_HC, _T), lambda i: (0, 0)),
            pl.BlockSpec((8, _T), lambda i: (0, 0)),
        ],
        out_specs=pl.BlockSpec((_G, _T), lambda i: (0, 0)),
        out_shape=jax.ShapeDtypeStruct((_G, _T), jnp.float32),
        scratch_shapes=[
            pltpu.VMEM((_G, _HC), jnp.float32),
            pltpu.VMEM((1, _G), jnp.float32),
        ],
    )(h, b2d, wh, bh8)


def _edge_sc_body(qp_hbm, kv_hbm, di_hbm, si_hbm, ea_hbm, z_hbm, acc_out,
                  di_a, di_b, si_a, si_b, ea_a, ea_b,
                  qp0, qp1, kv0, kv1, msg_v, acc_sh,
                  sq0, sq1, sk0, sk1, sdi_a, sdi_b, ssi_a, ssi_b,
                  sea_a, sea_b):
    cid = lax.axis_index("c")
    sid = lax.axis_index("s")
    wid = sid * _NC + cid
    row0 = sid * _RPT
    c0 = wid * _CPW

    pltpu.sync_copy(z_hbm.at[pl.ds(row0, _RPT)], acc_sh.at[pl.ds(row0, _RPT)])
    plsc.subcore_barrier()

    li = lax.iota(jnp.int32, 16)
    in_h = [(li >= 4 * h) & (li < 4 * h + 4) for h in range(_H)]
    qps = (qp0, qp1)
    kvs = (kv0, kv1)
    sqs = (sq0, sq1)
    sks = (sk0, sk1)
    dis = (di_a, di_b)
    sis = (si_a, si_b)
    eas = (ea_a, ea_b)
    sbsems = ((sdi_a, ssi_a, sea_a), (sdi_b, ssi_b, sea_b))

    def load_sb_sync(sb_global, buf):
        g0 = sb_global * _SB
        pltpu.sync_copy(di_hbm.at[pl.ds(g0, _SB)], dis[buf])
        pltpu.sync_copy(si_hbm.at[pl.ds(g0, _SB)], sis[buf])
        pltpu.sync_copy(ea_hbm.at[pl.ds(g0 * _CH, _SB * _CH)], eas[buf])

    def load_sb_async(sb_global, buf):
        g0 = sb_global * _SB
        pltpu.async_copy(di_hbm.at[pl.ds(g0, _SB)], dis[buf], sbsems[buf][0])
        pltpu.async_copy(si_hbm.at[pl.ds(g0, _SB)], sis[buf], sbsems[buf][1])
        pltpu.async_copy(ea_hbm.at[pl.ds(g0 * _CH, _SB * _CH)], eas[buf], sbsems[buf][2])

    def wait_sb(sb_global, buf):
        g0 = sb_global * _SB
        pltpu.make_async_copy(di_hbm.at[pl.ds(g0, _SB)], dis[buf], sbsems[buf][0]).wait()
        pltpu.make_async_copy(si_hbm.at[pl.ds(g0, _SB)], sis[buf], sbsems[buf][1]).wait()
        pltpu.make_async_copy(ea_hbm.at[pl.ds(g0 * _CH, _SB * _CH)], eas[buf], sbsems[buf][2]).wait()

    def issue_gathers(sbbuf, j, gbuf):
        pltpu.async_copy(qp_hbm.at[dis[sbbuf].at[j]], qps[gbuf], sqs[gbuf])
        pltpu.async_copy(kv_hbm.at[sis[sbbuf].at[j]], kvs[gbuf], sks[gbuf])

    def wait_gathers(sbbuf, j, gbuf):
        pltpu.make_async_copy(qp_hbm.at[dis[sbbuf].at[j]], qps[gbuf], sqs[gbuf]).wait()
        pltpu.make_async_copy(kv_hbm.at[sis[sbbuf].at[j]], kvs[gbuf], sks[gbuf]).wait()

    def compute_chunk(sbbuf, j, gbuf):
        qp_v = qps[gbuf]
        kv_v = kvs[gbuf]
        ea_v = eas[sbbuf]

        @plsc.parallel_loop(0, _CH, unroll=4)
        def edge(i):
            ea_row = ea_v[j * _CH + i, :]
            p_row = qp_v[i, pl.ds(_HC, 16)]
            pterm = p_row * ea_row
            exs = []
            for h in range(_H):
                sh = (qp_v[i, pl.ds(2 * h * 16, 16)] * kv_v[i, pl.ds(2 * h * 16, 16)]
                      + qp_v[i, pl.ds((2 * h + 1) * 16, 16)] * kv_v[i, pl.ds((2 * h + 1) * 16, 16)])
                sh = sh + jnp.where(in_h[h], pterm, 0.0)
                a_s = jnp.sum(sh)
                exs.append(jnp.exp(jnp.full((16,), a_s, jnp.float32)))
            for jj in range(8):
                msg_v[i, pl.ds(jj * 16, 16)] = (
                    kv_v[i, pl.ds(_HC + jj * 16, 16)] * exs[jj // 2])
            exrep = jnp.where(li < 4, exs[0],
                              jnp.where(li < 8, exs[1],
                                        jnp.where(li < 12, exs[2], exs[3])))
            msg_v[i, pl.ds(_HC, 16)] = exrep
            msg_v[i, pl.ds(_HC + _HE, 16)] = exrep * ea_row

        pltpu.sync_copy(msg_v, acc_sh.at[dis[sbbuf].at[j]], add=True)

    load_sb_sync(c0 // _SB, 0)
    issue_gathers(0, 0, 0)
    issue_gathers(0, 1, 1)

    def outer(s2, carry):
        sbA = c0 // _SB + 2 * s2
        sbB = sbA + 1
        u0 = s2 * 12
        for j in range(_SB):
            gbuf = j % 2
            if j == 0:
                load_sb_async(sbB, 1)
            if j == 4:
                wait_sb(sbB, 1)
            wait_gathers(0, j, gbuf)
            compute_chunk(0, j, gbuf)
            if j < _SB - 2:
                issue_gathers(0, j + 2, gbuf)
            else:
                issue_gathers(1, j + 2 - _SB, gbuf)
        for j in range(_SB):
            gbuf = j % 2
            if j == 0:
                @pl.when(s2 < _OUTER - 1)
                def _():
                    load_sb_async(sbA + 2, 0)
            if j == 4:
                @pl.when(s2 < _OUTER - 1)
                def _():
                    wait_sb(sbA + 2, 0)
            wait_gathers(1, j, gbuf)
            compute_chunk(1, j, gbuf)
            if j < _SB - 2:
                issue_gathers(1, j + 2, gbuf)
            else:
                @pl.when(s2 < _OUTER - 1)
                def _():
                    issue_gathers(0, j + 2 - _SB, gbuf)
        return carry

    lax.fori_loop(0, _OUTER, outer, 0)
    plsc.subcore_barrier()
    pltpu.sync_copy(acc_sh.at[pl.ds(row0, _RPT)],
                    acc_out.at[cid, pl.ds(row0, _RPT)])


@functools.cache
def _edge_kernel_built():
    mesh = plsc.VectorSubcoreMesh(core_axis_name="c", subcore_axis_name="s",
                                  num_cores=_NC, num_subcores=_NS)
    return pl.kernel(
        _edge_sc_body,
        out_type=jax.ShapeDtypeStruct((_NC, _NP, _ROW), jnp.float32),
        mesh=mesh,
        compiler_params=pltpu.CompilerParams(needs_layout_passes=False,
                                             use_tc_tiling_on_sc=False),
        scratch_types=[
            pltpu.VMEM((_SB, _CH), jnp.int32),
            pltpu.VMEM((_SB, _CH), jnp.int32),
            pltpu.VMEM((_SB, _CH), jnp.int32),
            pltpu.VMEM((_SB, _CH), jnp.int32),
            pltpu.VMEM((_SB * _CH, _HE), jnp.float32),
            pltpu.VMEM((_SB * _CH, _HE), jnp.float32),
            pltpu.VMEM((_CH, _HC + _HE), jnp.float32),
            pltpu.VMEM((_CH, _HC + _HE), jnp.float32),
            pltpu.VMEM((_CH, 2 * _HC), jnp.float32),
            pltpu.VMEM((_CH, 2 * _HC), jnp.float32),
            pltpu.VMEM((_CH, _ROW), jnp.float32),
            pltpu.VMEM_SHARED((_NP, _ROW), jnp.float32),
            pltpu.SemaphoreType.DMA,
            pltpu.SemaphoreType.DMA,
            pltpu.SemaphoreType.DMA,
            pltpu.SemaphoreType.DMA,
            pltpu.SemaphoreType.DMA,
            pltpu.SemaphoreType.DMA,
            pltpu.SemaphoreType.DMA,
            pltpu.SemaphoreType.DMA,
            pltpu.SemaphoreType.DMA,
            pltpu.SemaphoreType.DMA,
        ],
    )


def _edge_kernel(*args):
    return _edge_kernel_built()(*args)


def _layer(h, di2, si2, ea2, z, wq, bq, wk, bk, wv, bv, we, ws, bs, relu):
    wall = jnp.concatenate([wq * _INV_SQRT_C, wk, wv, ws], axis=1)
    ball8 = jnp.broadcast_to(
        jnp.concatenate([bq * _INV_SQRT_C, bk, bv, bs])[None, :], (8, 4 * _HC))
    wp = _WP_MASK * jnp.tile(we.T, (1, _H))
    qp, kv, s = _proj(h, wall, ball8, wp)
    acc = _edge_kernel(qp, kv, di2, si2, ea2, z)
    return _combine(acc, s, wp.T, relu)


def kernel(x, edge_index, edge_attr, batch, r_target,
           Wq1, bq1, Wk1, bk1, Wv1, bv1, We1, Ws1, bs1,
           Wq2, bq2, Wk2, bk2, Wv2, bv2, We2, Ws2, bs2,
           Wq3, bq3, Wk3, bk3, Wv3, bv3, We3, Ws3, bs3, Wh, bh):
    si = edge_index[0]
    di = edge_index[1]
    npad = _EP - _E
    di2 = jnp.concatenate([di, jnp.full((npad,), _NP - 1, jnp.int32)]).reshape(_NCHUNK, _CH)
    si2 = jnp.concatenate([si, jnp.zeros((npad,), jnp.int32)]).reshape(_NCHUNK, _CH)
    ea2 = jnp.concatenate([jnp.tile(edge_attr, (1, _H)),
                           jnp.zeros((npad, _HE), jnp.float32)])
    z = jnp.zeros((_NP, _ROW), jnp.float32)
    xp = jnp.concatenate([x, jnp.zeros((_NP - _N, _D), jnp.float32)])
    h = _layer(xp, di2, si2, ea2, z, Wq1, bq1, Wk1, bk1, Wv1, bv1, We1, Ws1, bs1, True)
    h = _layer(h, di2, si2, ea2, z, Wq2, bq2, Wk2, bk2, Wv2, bv2, We2, Ws2, bs2, True)
    h = _layer(h, di2, si2, ea2, z, Wq3, bq3, Wk3, bk3, Wv3, bv3, We3, Ws3, bs3, False)
    b2d = batch.astype(jnp.float32).reshape(_N, 1)
    bh8 = jnp.broadcast_to(bh[None, :], (8, _T))
    return _pool(h[:_N], b2d, Wh, bh8)


# speedup vs baseline: 9.4808x; 1.0299x over previous
import functools
import numpy as np
import jax
import jax.numpy as jnp
from jax import lax
from jax.experimental import pallas as pl
from jax.experimental.pallas import tpu as pltpu
from jax.experimental.pallas import tpu_sc as plsc

_N = 10000
_E = 320000
_D = 128
_H = 4
_C = 32
_ED = 4
_G = 128
_T = 4
_HC = _H * _C
_HE = _H * _ED
_ROW = 160
_NC = 2
_NS = 16
_NW = _NC * _NS
_NP = 10112
_RPT = _NP // _NS
_CH = 24
_EP = 322560
_NCHUNK = _EP // _CH
_CPW = _NCHUNK // _NW
_SB = 6
_OUTER = _CPW // (2 * _SB)
_INV_SQRT_C = float(1.0 / np.sqrt(_C))
_RB = 1000
_PRB = 632
_NBLK = _N // _RB

_WP_MASK = np.zeros((_HC, _HE), np.float32)
for _h in range(_H):
    _WP_MASK[_h * _C:(_h + 1) * _C, _h * _ED:(_h + 1) * _ED] = 1.0

_DEN_ONES2 = np.zeros((_HE, _HC), np.float32)
for _h in range(_H):
    _DEN_ONES2[4 * _h, _h * _C:(_h + 1) * _C] = 1.0


def _proj_body(x_ref, wall_ref, ball_ref, wp_ref, qp_ref, kv_ref, s_ref):
    x = x_ref[...]
    qkvs = jnp.dot(x, wall_ref[...], preferred_element_type=jnp.float32,
                   precision=lax.Precision.HIGHEST)
    qkvs = qkvs + ball_ref[0:1, :]
    q = qkvs[:, 0:_HC]
    p = jnp.dot(q, wp_ref[...], preferred_element_type=jnp.float32,
                precision=lax.Precision.HIGHEST)
    qp_ref[:, 0:_HC] = q
    qp_ref[:, _HC:_HC + _HE] = p
    kv_ref[...] = qkvs[:, _HC:3 * _HC]
    s_ref[...] = qkvs[:, 3 * _HC:4 * _HC]


def _proj(x, wall, ball8, wp):
    return pl.pallas_call(
        _proj_body,
        grid=(_NS,),
        in_specs=[
            pl.BlockSpec((_PRB, _D), lambda i: (i, 0)),
            pl.BlockSpec((_D, 4 * _HC), lambda i: (0, 0)),
            pl.BlockSpec((8, 4 * _HC), lambda i: (0, 0)),
            pl.BlockSpec((_HC, _HE), lambda i: (0, 0)),
        ],
        out_specs=[
            pl.BlockSpec((_PRB, _HC + _HE), lambda i: (i, 0)),
            pl.BlockSpec((_PRB, 2 * _HC), lambda i: (i, 0)),
            pl.BlockSpec((_PRB, _HC), lambda i: (i, 0)),
        ],
        out_shape=[
            jax.ShapeDtypeStruct((_NP, _HC + _HE), jnp.float32),
            jax.ShapeDtypeStruct((_NP, 2 * _HC), jnp.float32),
            jax.ShapeDtypeStruct((_NP, _HC), jnp.float32),
        ],
    )(x, wall, ball8, wp)


def _combine_body(acc_ref, s_ref, wse_ref, dones_ref, out_ref, *, relu):
    a = acc_ref[0] + acc_ref[1]
    num = a[:, 0:_HC]
    exr = a[:, _HC:_HC + _HE]
    s4 = a[:, _HC + _HE:_ROW]
    eterm = jnp.dot(s4, wse_ref[...], preferred_element_type=jnp.float32,
                    precision=lax.Precision.HIGHEST)
    den_rep = jnp.dot(exr, dones_ref[...], preferred_element_type=jnp.float32,
                      precision=lax.Precision.HIGHEST)
    h = (num + eterm) / (den_rep + 1e-16) + s_ref[...]
    if relu:
        h = jnp.maximum(h, 0.0)
    out_ref[...] = h


def _combine(acc, s, wse, relu):
    return pl.pallas_call(
        functools.partial(_combine_body, relu=relu),
        grid=(_NS,),
        in_specs=[
            pl.BlockSpec((_NC, _PRB, _ROW), lambda i: (0, i, 0)),
            pl.BlockSpec((_PRB, _HC), lambda i: (i, 0)),
            pl.BlockSpec((_HE, _HC), lambda i: (0, 0)),
            pl.BlockSpec((_HE, _HC), lambda i: (0, 0)),
        ],
        out_specs=pl.BlockSpec((_PRB, _HC), lambda i: (i, 0)),
        out_shape=jax.ShapeDtypeStruct((_NP, _HC), jnp.float32),
    )(acc, s, wse, _DEN_ONES2)


def _pool_body(h_ref, b_ref, wh_ref, bh_ref, out_ref, acc_ref, cnt_ref):
    i = pl.program_id(0)

    @pl.when(i == 0)
    def _init():
        acc_ref[...] = jnp.zeros_like(acc_ref)
        cnt_ref[...] = jnp.zeros_like(cnt_ref)

    b = b_ref[...]
    gid = lax.broadcasted_iota(jnp.int32, (_RB, _G), 1).astype(jnp.float32)
    onehot = (b == gid).astype(jnp.float32)
    acc_ref[...] += lax.dot_general(onehot, h_ref[...],
                                    (((0,), (0,)), ((), ())),
                                    preferred_element_type=jnp.float32,
                                    precision=lax.Precision.HIGHEST)
    cnt_ref[...] += jnp.sum(onehot, axis=0, keepdims=True)

    @pl.when(i == _NBLK - 1)
    def _final():
        cnt = jnp.maximum(cnt_ref[...], 1.0)
        pooled = acc_ref[...] / cnt.reshape(_G, 1)
        out_ref[...] = jnp.dot(pooled, wh_ref[...],
                               preferred_element_type=jnp.float32,
                               precision=lax.Precision.HIGHEST) + bh_ref[0:1, :]


def _pool(h, b2d, wh, bh8):
    return pl.pallas_call(
        _pool_body,
        grid=(_NBLK,),
        in_specs=[
            pl.BlockSpec((_RB, _HC), lambda i: (i, 0)),
            pl.BlockSpec((_RB, 1), lambda i: (i, 0)),
            pl.BlockSpec((_HC, _T), lambda i: (0, 0)),
            pl.BlockSpec((8, _T), lambda i: (0, 0)),
        ],
        out_specs=pl.BlockSpec((_G, _T), lambda i: (0, 0)),
        out_shape=jax.ShapeDtypeStruct((_G, _T), jnp.float32),
        scratch_shapes=[
            pltpu.VMEM((_G, _HC), jnp.float32),
            pltpu.VMEM((1, _G), jnp.float32),
        ],
    )(h, b2d, wh, bh8)


def _edge_sc_body(qp_hbm, kv_hbm, di_hbm, si_hbm, ea_hbm, z_hbm, acc_out,
                  di_a, di_b, si_a, si_b, ea_a, ea_b,
                  qp0, qp1, kv0, kv1, msg_v, acc_sh,
                  sq0, sq1, sk0, sk1, sdi_a, sdi_b, ssi_a, ssi_b,
                  sea_a, sea_b):
    cid = lax.axis_index("c")
    sid = lax.axis_index("s")
    wid = sid * _NC + cid
    row0 = sid * _RPT
    c0 = wid * _CPW

    pltpu.sync_copy(z_hbm.at[pl.ds(row0, _RPT)], acc_sh.at[pl.ds(row0, _RPT)])
    plsc.subcore_barrier()

    li = lax.iota(jnp.int32, 16)
    li4 = li % 4
    in_h = [(li >= 4 * h) & (li < 4 * h + 4) for h in range(_H)]
    qps = (qp0, qp1)
    kvs = (kv0, kv1)
    sqs = (sq0, sq1)
    sks = (sk0, sk1)
    dis = (di_a, di_b)
    sis = (si_a, si_b)
    eas = (ea_a, ea_b)
    sbsems = ((sdi_a, ssi_a, sea_a), (sdi_b, ssi_b, sea_b))

    def load_sb_sync(sb_global, buf):
        p0 = sb_global * (_SB // 2)
        pltpu.sync_copy(di_hbm.at[pl.ds(p0, _SB // 2)], dis[buf])
        pltpu.sync_copy(si_hbm.at[pl.ds(p0, _SB // 2)], sis[buf])
        pltpu.sync_copy(ea_hbm.at[pl.ds(p0 * 2 * _CH, _SB * _CH)], eas[buf])

    def load_sb_async(sb_global, buf):
        p0 = sb_global * (_SB // 2)
        pltpu.async_copy(di_hbm.at[pl.ds(p0, _SB // 2)], dis[buf], sbsems[buf][0])
        pltpu.async_copy(si_hbm.at[pl.ds(p0, _SB // 2)], sis[buf], sbsems[buf][1])
        pltpu.async_copy(ea_hbm.at[pl.ds(p0 * 2 * _CH, _SB * _CH)], eas[buf], sbsems[buf][2])

    def wait_sb(sb_global, buf):
        p0 = sb_global * (_SB // 2)
        pltpu.make_async_copy(di_hbm.at[pl.ds(p0, _SB // 2)], dis[buf], sbsems[buf][0]).wait()
        pltpu.make_async_copy(si_hbm.at[pl.ds(p0, _SB // 2)], sis[buf], sbsems[buf][1]).wait()
        pltpu.make_async_copy(ea_hbm.at[pl.ds(p0 * 2 * _CH, _SB * _CH)], eas[buf], sbsems[buf][2]).wait()

    def issue_gathers(sbbuf, j, gbuf):
        jp, half = j // 2, (j % 2) * _CH
        pltpu.async_copy(qp_hbm.at[dis[sbbuf].at[jp, pl.ds(half, _CH)]], qps[gbuf], sqs[gbuf])
        pltpu.async_copy(kv_hbm.at[sis[sbbuf].at[jp, pl.ds(half, _CH)]], kvs[gbuf], sks[gbuf])

    def wait_gathers(sbbuf, j, gbuf):
        jp, half = j // 2, (j % 2) * _CH
        pltpu.make_async_copy(qp_hbm.at[dis[sbbuf].at[jp, pl.ds(half, _CH)]], qps[gbuf], sqs[gbuf]).wait()
        pltpu.make_async_copy(kv_hbm.at[sis[sbbuf].at[jp, pl.ds(half, _CH)]], kvs[gbuf], sks[gbuf]).wait()

    def compute_chunk(sbbuf, j, gbuf):
        qp_v = qps[gbuf]
        kv_v = kvs[gbuf]
        ea_v = eas[sbbuf]

        half = (j % 2) * _CH

        @plsc.parallel_loop(0, _CH, unroll=8)
        def edge(i):
            ea_row = plsc.load_gather(
                ea_v, [jnp.full((16,), j * _CH + i, jnp.int32), li4])
            p_row = qp_v[i, pl.ds(_HC, 16)]
            pterm = p_row * ea_row
            exs = []
            for h in range(_H):
                sh = (qp_v[i, pl.ds(2 * h * 16, 16)] * kv_v[i, pl.ds(2 * h * 16, 16)]
                      + qp_v[i, pl.ds((2 * h + 1) * 16, 16)] * kv_v[i, pl.ds((2 * h + 1) * 16, 16)])
                sh = sh + jnp.where(in_h[h], pterm, 0.0)
                a_s = jnp.sum(sh)
                exs.append(jnp.exp(jnp.full((16,), a_s, jnp.float32)))
            for jj in range(8):
                msg_v[half + i, pl.ds(jj * 16, 16)] = (
                    kv_v[i, pl.ds(_HC + jj * 16, 16)] * exs[jj // 2])
            exrep = jnp.where(li < 4, exs[0],
                              jnp.where(li < 8, exs[1],
                                        jnp.where(li < 12, exs[2], exs[3])))
            msg_v[half + i, pl.ds(_HC, 16)] = exrep
            msg_v[half + i, pl.ds(_HC + _HE, 16)] = exrep * ea_row

        if j % 2 == 1:
            pltpu.sync_copy(msg_v, acc_sh.at[dis[sbbuf].at[j // 2]], add=True)

    load_sb_sync(c0 // _SB, 0)
    issue_gathers(0, 0, 0)
    issue_gathers(0, 1, 1)

    def outer(s2, carry):
        sbA = c0 // _SB + 2 * s2
        sbB = sbA + 1
        u0 = s2 * 12
        for j in range(_SB):
            gbuf = j % 2
            if j == 0:
                load_sb_async(sbB, 1)
            if j == 4:
                wait_sb(sbB, 1)
            wait_gathers(0, j, gbuf)
            compute_chunk(0, j, gbuf)
            if j < _SB - 2:
                issue_gathers(0, j + 2, gbuf)
            else:
                issue_gathers(1, j + 2 - _SB, gbuf)
        for j in range(_SB):
            gbuf = j % 2
            if j == 0:
                @pl.when(s2 < _OUTER - 1)
                def _():
                    load_sb_async(sbA + 2, 0)
            if j == 4:
                @pl.when(s2 < _OUTER - 1)
                def _():
                    wait_sb(sbA + 2, 0)
            wait_gathers(1, j, gbuf)
            compute_chunk(1, j, gbuf)
            if j < _SB - 2:
                issue_gathers(1, j + 2, gbuf)
            else:
                @pl.when(s2 < _OUTER - 1)
                def _():
                    issue_gathers(0, j + 2 - _SB, gbuf)
        return carry

    lax.fori_loop(0, _OUTER, outer, 0)
    plsc.subcore_barrier()
    pltpu.sync_copy(acc_sh.at[pl.ds(row0, _RPT)],
                    acc_out.at[cid, pl.ds(row0, _RPT)])


@functools.cache
def _edge_kernel_built():
    mesh = plsc.VectorSubcoreMesh(core_axis_name="c", subcore_axis_name="s",
                                  num_cores=_NC, num_subcores=_NS)
    return pl.kernel(
        _edge_sc_body,
        out_type=jax.ShapeDtypeStruct((_NC, _NP, _ROW), jnp.float32),
        mesh=mesh,
        compiler_params=pltpu.CompilerParams(needs_layout_passes=False,
                                             use_tc_tiling_on_sc=False),
        scratch_types=[
            pltpu.VMEM((_SB // 2, 2 * _CH), jnp.int32),
            pltpu.VMEM((_SB // 2, 2 * _CH), jnp.int32),
            pltpu.VMEM((_SB // 2, 2 * _CH), jnp.int32),
            pltpu.VMEM((_SB // 2, 2 * _CH), jnp.int32),
            pltpu.VMEM((_SB * _CH, _ED), jnp.float32),
            pltpu.VMEM((_SB * _CH, _ED), jnp.float32),
            pltpu.VMEM((_CH, _HC + _HE), jnp.float32),
            pltpu.VMEM((_CH, _HC + _HE), jnp.float32),
            pltpu.VMEM((_CH, 2 * _HC), jnp.float32),
            pltpu.VMEM((_CH, 2 * _HC), jnp.float32),
            pltpu.VMEM((2 * _CH, _ROW), jnp.float32),
            pltpu.VMEM_SHARED((_NP, _ROW), jnp.float32),
            pltpu.SemaphoreType.DMA,
            pltpu.SemaphoreType.DMA,
            pltpu.SemaphoreType.DMA,
            pltpu.SemaphoreType.DMA,
            pltpu.SemaphoreType.DMA,
            pltpu.SemaphoreType.DMA,
            pltpu.SemaphoreType.DMA,
            pltpu.SemaphoreType.DMA,
            pltpu.SemaphoreType.DMA,
            pltpu.SemaphoreType.DMA,
        ],
    )


def _edge_kernel(*args):
    return _edge_kernel_built()(*args)


def _layer(h, di2, si2, ea2, z, wq, bq, wk, bk, wv, bv, we, ws, bs, relu):
    wall = jnp.concatenate([wq * _INV_SQRT_C, wk, wv, ws], axis=1)
    ball8 = jnp.broadcast_to(
        jnp.concatenate([bq * _INV_SQRT_C, bk, bv, bs])[None, :], (8, 4 * _HC))
    wp = _WP_MASK * jnp.tile(we.T, (1, _H))
    qp, kv, s = _proj(h, wall, ball8, wp)
    acc = _edge_kernel(qp, kv, di2, si2, ea2, z)
    return _combine(acc, s, wp.T, relu)


def kernel(x, edge_index, edge_attr, batch, r_target,
           Wq1, bq1, Wk1, bk1, Wv1, bv1, We1, Ws1, bs1,
           Wq2, bq2, Wk2, bk2, Wv2, bv2, We2, Ws2, bs2,
           Wq3, bq3, Wk3, bk3, Wv3, bv3, We3, Ws3, bs3, Wh, bh):
    si = edge_index[0]
    di = edge_index[1]
    npad = _EP - _E
    di2 = jnp.concatenate([di, jnp.full((npad,), _NP - 1, jnp.int32)]).reshape(_NCHUNK // 2, 2 * _CH)
    si2 = jnp.concatenate([si, jnp.zeros((npad,), jnp.int32)]).reshape(_NCHUNK // 2, 2 * _CH)
    ea2 = jnp.concatenate([edge_attr,
                           jnp.zeros((npad, _ED), jnp.float32)])
    z = jnp.zeros((_NP, _ROW), jnp.float32)
    xp = jnp.concatenate([x, jnp.zeros((_NP - _N, _D), jnp.float32)])
    h = _layer(xp, di2, si2, ea2, z, Wq1, bq1, Wk1, bk1, Wv1, bv1, We1, Ws1, bs1, True)
    h = _layer(h, di2, si2, ea2, z, Wq2, bq2, Wk2, bk2, Wv2, bv2, We2, Ws2, bs2, True)
    h = _layer(h, di2, si2, ea2, z, Wq3, bq3, Wk3, bk3, Wv3, bv3, We3, Ws3, bs3, False)
    b2d = batch.astype(jnp.float32).reshape(_N, 1)
    bh8 = jnp.broadcast_to(bh[None, :], (8, _T))
    return _pool(h[:_N], b2d, Wh, bh8)


# speedup vs baseline: 9.5684x; 1.0092x over previous
import functools
import numpy as np
import jax
import jax.numpy as jnp
from jax import lax
from jax.experimental import pallas as pl
from jax.experimental.pallas import tpu as pltpu
from jax.experimental.pallas import tpu_sc as plsc

_N = 10000
_E = 320000
_D = 128
_H = 4
_C = 32
_ED = 4
_G = 128
_T = 4
_HC = _H * _C
_HE = _H * _ED
_ROW = 160
_NC = 2
_NS = 16
_NW = _NC * _NS
_NP = 10112
_RPT = _NP // _NS
_CH = 24
_EP = 322560
_NCHUNK = _EP // _CH
_CPW = _NCHUNK // _NW
_SB = 6
_OUTER = _CPW // (2 * _SB)
_INV_SQRT_C = float(1.0 / np.sqrt(_C))
_RB = 1000
_PRB = 632
_NBLK = _N // _RB

_WP_MASK = np.zeros((_HC, _HE), np.float32)
for _h in range(_H):
    _WP_MASK[_h * _C:(_h + 1) * _C, _h * _ED:(_h + 1) * _ED] = 1.0

_DEN_ONES2 = np.zeros((_HE, _HC), np.float32)
for _h in range(_H):
    _DEN_ONES2[4 * _h, _h * _C:(_h + 1) * _C] = 1.0


def _proj_body(x_ref, wall_ref, ball_ref, wp_ref, qp_ref, kv_ref, s_ref):
    x = x_ref[...]
    qkvs = jnp.dot(x, wall_ref[...], preferred_element_type=jnp.float32,
                   precision=lax.Precision.HIGHEST)
    qkvs = qkvs + ball_ref[0:1, :]
    q = qkvs[:, 0:_HC]
    p = jnp.dot(q, wp_ref[...], preferred_element_type=jnp.float32,
                precision=lax.Precision.HIGHEST)
    qp_ref[:, 0:_HC] = q
    qp_ref[:, _HC:_HC + _HE] = p
    kv_ref[...] = qkvs[:, _HC:3 * _HC]
    s_ref[...] = qkvs[:, 3 * _HC:4 * _HC]


def _proj(x, wall, ball8, wp):
    return pl.pallas_call(
        _proj_body,
        grid=(_NS,),
        in_specs=[
            pl.BlockSpec((_PRB, _D), lambda i: (i, 0)),
            pl.BlockSpec((_D, 4 * _HC), lambda i: (0, 0)),
            pl.BlockSpec((8, 4 * _HC), lambda i: (0, 0)),
            pl.BlockSpec((_HC, _HE), lambda i: (0, 0)),
        ],
        out_specs=[
            pl.BlockSpec((_PRB, _HC + _HE), lambda i: (i, 0)),
            pl.BlockSpec((_PRB, 2 * _HC), lambda i: (i, 0)),
            pl.BlockSpec((_PRB, _HC), lambda i: (i, 0)),
        ],
        out_shape=[
            jax.ShapeDtypeStruct((_NP, _HC + _HE), jnp.float32),
            jax.ShapeDtypeStruct((_NP, 2 * _HC), jnp.float32),
            jax.ShapeDtypeStruct((_NP, _HC), jnp.float32),
        ],
    )(x, wall, ball8, wp)


def _combine_body(acc_ref, s_ref, wse_ref, dones_ref, out_ref, *, relu):
    a = acc_ref[0] + acc_ref[1]
    num = a[:, 0:_HC]
    exr = a[:, _HC:_HC + _HE]
    s4 = a[:, _HC + _HE:_ROW]
    eterm = jnp.dot(s4, wse_ref[...], preferred_element_type=jnp.float32,
                    precision=lax.Precision.HIGHEST)
    den_rep = jnp.dot(exr, dones_ref[...], preferred_element_type=jnp.float32,
                      precision=lax.Precision.HIGHEST)
    h = (num + eterm) / (den_rep + 1e-16) + s_ref[...]
    if relu:
        h = jnp.maximum(h, 0.0)
    out_ref[...] = h


def _combine(acc, s, wse, relu):
    return pl.pallas_call(
        functools.partial(_combine_body, relu=relu),
        grid=(_NS,),
        in_specs=[
            pl.BlockSpec((_NC, _PRB, _ROW), lambda i: (0, i, 0)),
            pl.BlockSpec((_PRB, _HC), lambda i: (i, 0)),
            pl.BlockSpec((_HE, _HC), lambda i: (0, 0)),
            pl.BlockSpec((_HE, _HC), lambda i: (0, 0)),
        ],
        out_specs=pl.BlockSpec((_PRB, _HC), lambda i: (i, 0)),
        out_shape=jax.ShapeDtypeStruct((_NP, _HC), jnp.float32),
    )(acc, s, wse, _DEN_ONES2)


def _pool_body(h_ref, b_ref, wh_ref, bh_ref, out_ref, acc_ref, cnt_ref):
    i = pl.program_id(0)

    @pl.when(i == 0)
    def _init():
        acc_ref[...] = jnp.zeros_like(acc_ref)
        cnt_ref[...] = jnp.zeros_like(cnt_ref)

    b = b_ref[...]
    gid = lax.broadcasted_iota(jnp.int32, (_RB, _G), 1).astype(jnp.float32)
    onehot = (b == gid).astype(jnp.float32)
    acc_ref[...] += lax.dot_general(onehot, h_ref[...],
                                    (((0,), (0,)), ((), ())),
                                    preferred_element_type=jnp.float32,
                                    precision=lax.Precision.HIGHEST)
    cnt_ref[...] += jnp.sum(onehot, axis=0, keepdims=True)

    @pl.when(i == _NBLK - 1)
    def _final():
        cnt = jnp.maximum(cnt_ref[...], 1.0)
        pooled = acc_ref[...] / cnt.reshape(_G, 1)
        out_ref[...] = jnp.dot(pooled, wh_ref[...],
                               preferred_element_type=jnp.float32,
                               precision=lax.Precision.HIGHEST) + bh_ref[0:1, :]


def _pool(h, b2d, wh, bh8):
    return pl.pallas_call(
        _pool_body,
        grid=(_NBLK,),
        in_specs=[
            pl.BlockSpec((_RB, _HC), lambda i: (i, 0)),
            pl.BlockSpec((_RB, 1), lambda i: (i, 0)),
            pl.BlockSpec((_HC, _T), lambda i: (0, 0)),
            pl.BlockSpec((8, _T), lambda i: (0, 0)),
        ],
        out_specs=pl.BlockSpec((_G, _T), lambda i: (0, 0)),
        out_shape=jax.ShapeDtypeStruct((_G, _T), jnp.float32),
        scratch_shapes=[
            pltpu.VMEM((_G, _HC), jnp.float32),
            pltpu.VMEM((1, _G), jnp.float32),
        ],
    )(h, b2d, wh, bh8)


def _fused_body(acc_ref, s_ref, wse_ref, dones_ref, wall_ref, ball_ref,
                wp_ref, qp_ref, kv_ref, snext_ref):
    a = acc_ref[0] + acc_ref[1]
    num = a[:, 0:_HC]
    exr = a[:, _HC:_HC + _HE]
    s4 = a[:, _HC + _HE:_ROW]
    eterm = jnp.dot(s4, wse_ref[...], preferred_element_type=jnp.float32,
                    precision=lax.Precision.HIGHEST)
    den_rep = jnp.dot(exr, dones_ref[...], preferred_element_type=jnp.float32,
                      precision=lax.Precision.HIGHEST)
    h = jnp.maximum((num + eterm) / (den_rep + 1e-16) + s_ref[...], 0.0)
    qkvs = jnp.dot(h, wall_ref[...], preferred_element_type=jnp.float32,
                   precision=lax.Precision.HIGHEST)
    qkvs = qkvs + ball_ref[0:1, :]
    q = qkvs[:, 0:_HC]
    p = jnp.dot(q, wp_ref[...], preferred_element_type=jnp.float32,
                precision=lax.Precision.HIGHEST)
    qp_ref[:, 0:_HC] = q
    qp_ref[:, _HC:_HC + _HE] = p
    kv_ref[...] = qkvs[:, _HC:3 * _HC]
    snext_ref[...] = qkvs[:, 3 * _HC:4 * _HC]


def _fused(acc, s, wse, wall, ball8, wp):
    return pl.pallas_call(
        _fused_body,
        grid=(_NS,),
        in_specs=[
            pl.BlockSpec((_NC, _PRB, _ROW), lambda i: (0, i, 0)),
            pl.BlockSpec((_PRB, _HC), lambda i: (i, 0)),
            pl.BlockSpec((_HE, _HC), lambda i: (0, 0)),
            pl.BlockSpec((_HE, _HC), lambda i: (0, 0)),
            pl.BlockSpec((_D, 4 * _HC), lambda i: (0, 0)),
            pl.BlockSpec((8, 4 * _HC), lambda i: (0, 0)),
            pl.BlockSpec((_HC, _HE), lambda i: (0, 0)),
        ],
        out_specs=[
            pl.BlockSpec((_PRB, _HC + _HE), lambda i: (i, 0)),
            pl.BlockSpec((_PRB, 2 * _HC), lambda i: (i, 0)),
            pl.BlockSpec((_PRB, _HC), lambda i: (i, 0)),
        ],
        out_shape=[
            jax.ShapeDtypeStruct((_NP, _HC + _HE), jnp.float32),
            jax.ShapeDtypeStruct((_NP, 2 * _HC), jnp.float32),
            jax.ShapeDtypeStruct((_NP, _HC), jnp.float32),
        ],
    )(acc, s, wse, _DEN_ONES2, wall, ball8, wp)


def _cpool_body(acc_ref, s_ref, wse_ref, dones_ref, b_ref, wh_ref, bh_ref,
                out_ref, accp_ref, cnt_ref):
    i = pl.program_id(0)

    @pl.when(i == 0)
    def _init():
        accp_ref[...] = jnp.zeros_like(accp_ref)
        cnt_ref[...] = jnp.zeros_like(cnt_ref)

    a = acc_ref[0] + acc_ref[1]
    num = a[:, 0:_HC]
    exr = a[:, _HC:_HC + _HE]
    s4 = a[:, _HC + _HE:_ROW]
    eterm = jnp.dot(s4, wse_ref[...], preferred_element_type=jnp.float32,
                    precision=lax.Precision.HIGHEST)
    den_rep = jnp.dot(exr, dones_ref[...], preferred_element_type=jnp.float32,
                      precision=lax.Precision.HIGHEST)
    h = (num + eterm) / (den_rep + 1e-16) + s_ref[...]
    b = b_ref[...]
    gid = lax.broadcasted_iota(jnp.int32, (_PRB, _G), 1).astype(jnp.float32)
    onehot = (b == gid).astype(jnp.float32)
    accp_ref[...] += lax.dot_general(onehot, h,
                                     (((0,), (0,)), ((), ())),
                                     preferred_element_type=jnp.float32,
                                     precision=lax.Precision.HIGHEST)
    cnt_ref[...] += jnp.sum(onehot, axis=0, keepdims=True)

    @pl.when(i == _NS - 1)
    def _final():
        cnt = jnp.maximum(cnt_ref[...], 1.0)
        pooled = accp_ref[...] / cnt.reshape(_G, 1)
        out_ref[...] = jnp.dot(pooled, wh_ref[...],
                               preferred_element_type=jnp.float32,
                               precision=lax.Precision.HIGHEST) + bh_ref[0:1, :]


def _cpool(acc, s, wse, b2d, wh, bh8):
    return pl.pallas_call(
        _cpool_body,
        grid=(_NS,),
        in_specs=[
            pl.BlockSpec((_NC, _PRB, _ROW), lambda i: (0, i, 0)),
            pl.BlockSpec((_PRB, _HC), lambda i: (i, 0)),
            pl.BlockSpec((_HE, _HC), lambda i: (0, 0)),
            pl.BlockSpec((_HE, _HC), lambda i: (0, 0)),
            pl.BlockSpec((_PRB, 1), lambda i: (i, 0)),
            pl.BlockSpec((_HC, _T), lambda i: (0, 0)),
            pl.BlockSpec((8, _T), lambda i: (0, 0)),
        ],
        out_specs=pl.BlockSpec((_G, _T), lambda i: (0, 0)),
        out_shape=jax.ShapeDtypeStruct((_G, _T), jnp.float32),
        scratch_shapes=[
            pltpu.VMEM((_G, _HC), jnp.float32),
            pltpu.VMEM((1, _G), jnp.float32),
        ],
    )(acc, s, wse, _DEN_ONES2, b2d, wh, bh8)


def _edge_sc_body(qp_hbm, kv_hbm, di_hbm, si_hbm, ea_hbm, z_hbm, acc_out,
                  di_a, di_b, si_a, si_b, ea_a, ea_b,
                  qp0, qp1, kv0, kv1, msg_v, acc_sh,
                  sq0, sq1, sk0, sk1, sdi_a, sdi_b, ssi_a, ssi_b,
                  sea_a, sea_b):
    cid = lax.axis_index("c")
    sid = lax.axis_index("s")
    wid = sid * _NC + cid
    row0 = sid * _RPT
    c0 = wid * _CPW

    pltpu.sync_copy(z_hbm.at[pl.ds(row0, _RPT)], acc_sh.at[pl.ds(row0, _RPT)])
    plsc.subcore_barrier()

    li = lax.iota(jnp.int32, 16)
    li4 = li % 4
    in_h = [(li >= 4 * h) & (li < 4 * h + 4) for h in range(_H)]
    qps = (qp0, qp1)
    kvs = (kv0, kv1)
    sqs = (sq0, sq1)
    sks = (sk0, sk1)
    dis = (di_a, di_b)
    sis = (si_a, si_b)
    eas = (ea_a, ea_b)
    sbsems = ((sdi_a, ssi_a, sea_a), (sdi_b, ssi_b, sea_b))

    def load_sb_sync(sb_global, buf):
        p0 = sb_global * (_SB // 2)
        pltpu.sync_copy(di_hbm.at[pl.ds(p0, _SB // 2)], dis[buf])
        pltpu.sync_copy(si_hbm.at[pl.ds(p0, _SB // 2)], sis[buf])
        pltpu.sync_copy(ea_hbm.at[pl.ds(p0 * 2 * _CH, _SB * _CH)], eas[buf])

    def load_sb_async(sb_global, buf):
        p0 = sb_global * (_SB // 2)
        pltpu.async_copy(di_hbm.at[pl.ds(p0, _SB // 2)], dis[buf], sbsems[buf][0])
        pltpu.async_copy(si_hbm.at[pl.ds(p0, _SB // 2)], sis[buf], sbsems[buf][1])
        pltpu.async_copy(ea_hbm.at[pl.ds(p0 * 2 * _CH, _SB * _CH)], eas[buf], sbsems[buf][2])

    def wait_sb(sb_global, buf):
        p0 = sb_global * (_SB // 2)
        pltpu.make_async_copy(di_hbm.at[pl.ds(p0, _SB // 2)], dis[buf], sbsems[buf][0]).wait()
        pltpu.make_async_copy(si_hbm.at[pl.ds(p0, _SB // 2)], sis[buf], sbsems[buf][1]).wait()
        pltpu.make_async_copy(ea_hbm.at[pl.ds(p0 * 2 * _CH, _SB * _CH)], eas[buf], sbsems[buf][2]).wait()

    def issue_gathers(sbbuf, j, gbuf):
        jp, half = j // 2, (j % 2) * _CH
        pltpu.async_copy(qp_hbm.at[dis[sbbuf].at[jp, pl.ds(half, _CH)]], qps[gbuf], sqs[gbuf])
        pltpu.async_copy(kv_hbm.at[sis[sbbuf].at[jp, pl.ds(half, _CH)]], kvs[gbuf], sks[gbuf])

    def wait_gathers(sbbuf, j, gbuf):
        jp, half = j // 2, (j % 2) * _CH
        pltpu.make_async_copy(qp_hbm.at[dis[sbbuf].at[jp, pl.ds(half, _CH)]], qps[gbuf], sqs[gbuf]).wait()
        pltpu.make_async_copy(kv_hbm.at[sis[sbbuf].at[jp, pl.ds(half, _CH)]], kvs[gbuf], sks[gbuf]).wait()

    def compute_chunk(sbbuf, j, gbuf):
        qp_v = qps[gbuf]
        kv_v = kvs[gbuf]
        ea_v = eas[sbbuf]

        half = (j % 2) * _CH

        @plsc.parallel_loop(0, _CH, unroll=8)
        def edge(i):
            ea_row = plsc.load_gather(
                ea_v, [jnp.full((16,), j * _CH + i, jnp.int32), li4])
            p_row = qp_v[i, pl.ds(_HC, 16)]
            pterm = p_row * ea_row
            exs = []
            for h in range(_H):
                sh = (qp_v[i, pl.ds(2 * h * 16, 16)] * kv_v[i, pl.ds(2 * h * 16, 16)]
                      + qp_v[i, pl.ds((2 * h + 1) * 16, 16)] * kv_v[i, pl.ds((2 * h + 1) * 16, 16)])
                sh = sh + jnp.where(in_h[h], pterm, 0.0)
                a_s = jnp.sum(sh)
                exs.append(jnp.exp(jnp.full((16,), a_s, jnp.float32)))
            for jj in range(8):
                msg_v[half + i, pl.ds(jj * 16, 16)] = (
                    kv_v[i, pl.ds(_HC + jj * 16, 16)] * exs[jj // 2])
            exrep = jnp.where(li < 4, exs[0],
                              jnp.where(li < 8, exs[1],
                                        jnp.where(li < 12, exs[2], exs[3])))
            msg_v[half + i, pl.ds(_HC, 16)] = exrep
            msg_v[half + i, pl.ds(_HC + _HE, 16)] = exrep * ea_row

        if j % 2 == 1:
            pltpu.sync_copy(msg_v, acc_sh.at[dis[sbbuf].at[j // 2]], add=True)

    load_sb_sync(c0 // _SB, 0)
    issue_gathers(0, 0, 0)
    issue_gathers(0, 1, 1)

    def outer(s2, carry):
        sbA = c0 // _SB + 2 * s2
        sbB = sbA + 1
        u0 = s2 * 12
        for j in range(_SB):
            gbuf = j % 2
            if j == 0:
                load_sb_async(sbB, 1)
            if j == 4:
                wait_sb(sbB, 1)
            wait_gathers(0, j, gbuf)
            compute_chunk(0, j, gbuf)
            if j < _SB - 2:
                issue_gathers(0, j + 2, gbuf)
            else:
                issue_gathers(1, j + 2 - _SB, gbuf)
        for j in range(_SB):
            gbuf = j % 2
            if j == 0:
                @pl.when(s2 < _OUTER - 1)
                def _():
                    load_sb_async(sbA + 2, 0)
            if j == 4:
                @pl.when(s2 < _OUTER - 1)
                def _():
                    wait_sb(sbA + 2, 0)
            wait_gathers(1, j, gbuf)
            compute_chunk(1, j, gbuf)
            if j < _SB - 2:
                issue_gathers(1, j + 2, gbuf)
            else:
                @pl.when(s2 < _OUTER - 1)
                def _():
                    issue_gathers(0, j + 2 - _SB, gbuf)
        return carry

    lax.fori_loop(0, _OUTER, outer, 0)
    plsc.subcore_barrier()
    pltpu.sync_copy(acc_sh.at[pl.ds(row0, _RPT)],
                    acc_out.at[cid, pl.ds(row0, _RPT)])


@functools.cache
def _edge_kernel_built():
    mesh = plsc.VectorSubcoreMesh(core_axis_name="c", subcore_axis_name="s",
                                  num_cores=_NC, num_subcores=_NS)
    return pl.kernel(
        _edge_sc_body,
        out_type=jax.ShapeDtypeStruct((_NC, _NP, _ROW), jnp.float32),
        mesh=mesh,
        compiler_params=pltpu.CompilerParams(needs_layout_passes=False,
                                             use_tc_tiling_on_sc=False),
        scratch_types=[
            pltpu.VMEM((_SB // 2, 2 * _CH), jnp.int32),
            pltpu.VMEM((_SB // 2, 2 * _CH), jnp.int32),
            pltpu.VMEM((_SB // 2, 2 * _CH), jnp.int32),
            pltpu.VMEM((_SB // 2, 2 * _CH), jnp.int32),
            pltpu.VMEM((_SB * _CH, _ED), jnp.float32),
            pltpu.VMEM((_SB * _CH, _ED), jnp.float32),
            pltpu.VMEM((_CH, _HC + _HE), jnp.float32),
            pltpu.VMEM((_CH, _HC + _HE), jnp.float32),
            pltpu.VMEM((_CH, 2 * _HC), jnp.float32),
            pltpu.VMEM((_CH, 2 * _HC), jnp.float32),
            pltpu.VMEM((2 * _CH, _ROW), jnp.float32),
            pltpu.VMEM_SHARED((_NP, _ROW), jnp.float32),
            pltpu.SemaphoreType.DMA,
            pltpu.SemaphoreType.DMA,
            pltpu.SemaphoreType.DMA,
            pltpu.SemaphoreType.DMA,
            pltpu.SemaphoreType.DMA,
            pltpu.SemaphoreType.DMA,
            pltpu.SemaphoreType.DMA,
            pltpu.SemaphoreType.DMA,
            pltpu.SemaphoreType.DMA,
            pltpu.SemaphoreType.DMA,
        ],
    )


def _edge_kernel(*args):
    return _edge_kernel_built()(*args)


def _wparams(wq, bq, wk, bk, wv, bv, we, ws, bs):
    wall = jnp.concatenate([wq * _INV_SQRT_C, wk, wv, ws], axis=1)
    ball8 = jnp.broadcast_to(
        jnp.concatenate([bq * _INV_SQRT_C, bk, bv, bs])[None, :], (8, 4 * _HC))
    wp = _WP_MASK * jnp.tile(we.T, (1, _H))
    return wall, ball8, wp


def kernel(x, edge_index, edge_attr, batch, r_target,
           Wq1, bq1, Wk1, bk1, Wv1, bv1, We1, Ws1, bs1,
           Wq2, bq2, Wk2, bk2, Wv2, bv2, We2, Ws2, bs2,
           Wq3, bq3, Wk3, bk3, Wv3, bv3, We3, Ws3, bs3, Wh, bh):
    si = edge_index[0]
    di = edge_index[1]
    npad = _EP - _E
    di2 = jnp.concatenate([di, jnp.full((npad,), _NP - 1, jnp.int32)]).reshape(_NCHUNK // 2, 2 * _CH)
    si2 = jnp.concatenate([si, jnp.zeros((npad,), jnp.int32)]).reshape(_NCHUNK // 2, 2 * _CH)
    ea2 = jnp.concatenate([edge_attr,
                           jnp.zeros((npad, _ED), jnp.float32)])
    z = jnp.zeros((_NP, _ROW), jnp.float32)
    xp = jnp.concatenate([x, jnp.zeros((_NP - _N, _D), jnp.float32)])
    w1 = _wparams(Wq1, bq1, Wk1, bk1, Wv1, bv1, We1, Ws1, bs1)
    w2 = _wparams(Wq2, bq2, Wk2, bk2, Wv2, bv2, We2, Ws2, bs2)
    w3 = _wparams(Wq3, bq3, Wk3, bk3, Wv3, bv3, We3, Ws3, bs3)
    qp, kv, s = _proj(xp, *w1)
    acc = _edge_kernel(qp, kv, di2, si2, ea2, z)
    qp, kv, s = _fused(acc, s, w1[2].T, w2[0], w2[1], w2[2])
    acc = _edge_kernel(qp, kv, di2, si2, ea2, z)
    qp, kv, s = _fused(acc, s, w2[2].T, w3[0], w3[1], w3[2])
    acc = _edge_kernel(qp, kv, di2, si2, ea2, z)
    b2d = jnp.concatenate([batch.astype(jnp.float32),
                           jnp.full((_NP - _N,), -1.0, jnp.float32)]).reshape(_NP, 1)
    bh8 = jnp.broadcast_to(bh[None, :], (8, _T))
    return _cpool(acc, s, w3[2].T, b2d, Wh, bh8)
